# Initial kernel scaffold; baseline (speedup 1.0000x reference)
import jax
import jax.numpy as jnp
from jax.experimental import pallas as pl


def kernel(x):
    raise NotImplementedError("write your pallas kernel here")


# speedup vs baseline: 1.0047x; 1.0047x over previous
import functools

import jax
import jax.numpy as jnp
from jax import lax
from jax.experimental import pallas as pl
from jax.experimental.pallas import tpu as pltpu

_EPS = 1e-5
_TB = 8


def _ln_row_kernel(x_ref, o_ref, *, inv_n):
    x = x_ref[...]
    s = jnp.sum(x, axis=1, keepdims=True)
    q = jnp.sum(x * x, axis=1, keepdims=True)
    mean = s * inv_n
    var = jnp.maximum(q * inv_n - mean * mean, 0.0)
    scale = lax.rsqrt(var + _EPS)
    shift = -mean * scale
    o_ref[...] = x * scale + shift


def kernel(x):
    b = int(x.shape[0])
    n = 1
    for d in x.shape[1:]:
        n *= int(d)

    tb = _TB
    while b % tb:
        tb //= 2

    x2 = x.reshape(b, n)
    out = pl.pallas_call(
        functools.partial(_ln_row_kernel, inv_n=1.0 / float(n)),
        out_shape=jax.ShapeDtypeStruct((b, n), x.dtype),
        grid=(b // tb,),
        in_specs=[pl.BlockSpec((tb, n), lambda i: (i, 0))],
        out_specs=pl.BlockSpec((tb, n), lambda i: (i, 0)),
        compiler_params=pltpu.CompilerParams(
            dimension_semantics=("parallel",),
            vmem_limit_bytes=50 * 1024 * 1024,
        ),
        cost_estimate=pl.CostEstimate(
            flops=7 * b * n,
            transcendentals=b,
            bytes_accessed=2 * b * n * 4,
        ),
    )(x2)
    return out.reshape(x.shape)


# speedup vs baseline: 1.6162x; 1.6087x over previous
import functools

import jax
import jax.numpy as jnp
from jax import lax
from jax.experimental import pallas as pl
from jax.experimental.pallas import tpu as pltpu

_EPS = 1e-5
_TB = 8


def _ln_row_kernel(x_ref, o_ref, *, inv_n):
    x = x_ref[...]
    s1 = jnp.sum(x, axis=1, keepdims=True)
    q1 = jnp.sum(x * x, axis=1, keepdims=True)
    s = jnp.sum(s1, axis=2, keepdims=True)
    q = jnp.sum(q1, axis=2, keepdims=True)
    mean = s * inv_n
    var = jnp.maximum(q * inv_n - mean * mean, 0.0)
    scale = lax.rsqrt(var + _EPS)
    shift = -mean * scale
    o_ref[...] = x * scale + shift


def kernel(x):
    b = int(x.shape[0])
    s = int(x.shape[1])
    n = 1
    for d in x.shape[1:]:
        n *= int(d)
    c = n // s

    tb = _TB
    while b % tb:
        tb //= 2

    x3 = x.reshape(b, s, c)
    out = pl.pallas_call(
        functools.partial(_ln_row_kernel, inv_n=1.0 / float(n)),
        out_shape=jax.ShapeDtypeStruct((b, s, c), x.dtype),
        grid=(b // tb,),
        in_specs=[pl.BlockSpec((tb, s, c), lambda i: (i, 0, 0))],
        out_specs=pl.BlockSpec((tb, s, c), lambda i: (i, 0, 0)),
        compiler_params=pltpu.CompilerParams(
            dimension_semantics=("parallel",),
            vmem_limit_bytes=50 * 1024 * 1024,
        ),
        cost_estimate=pl.CostEstimate(
            flops=7 * b * n,
            transcendentals=b,
            bytes_accessed=2 * b * n * 4,
        ),
    )(x3)
    return out.reshape(x.shape)
